# speedup vs baseline: 1.8446x; 1.8446x over previous
import jax
import jax.numpy as jnp
from jax.experimental import pallas as pl

_B = 8192
_A = 154
_H = 768
_R = 1024


def _body(rep_ref, x_ref, wih_ref, whh_ref, bih_ref, bhh_ref,
          whead_ref, bhead_ref, emb_ref, p1_ref, p2_ref):
    rep = rep_ref[...]
    xm = x_ref[...]
    b_ih = bih_ref[...]
    b_hh = bhh_ref[...]
    H = _H

    gi = jnp.dot(rep, wih_ref[...], preferred_element_type=jnp.float32) + b_ih
    r = jax.nn.sigmoid(gi[:, :H] + b_hh[:, :H])
    z = jax.nn.sigmoid(gi[:, H:2 * H] + b_hh[:, H:2 * H])
    n = jnp.tanh(gi[:, 2 * H:] + r * b_hh[:, 2 * H:])
    h1 = (1.0 - z) * n

    out1 = jnp.dot(h1, whead_ref[...], preferred_element_type=jnp.float32) + bhead_ref[...]

    cols = jax.lax.broadcasted_iota(jnp.int32, (rep.shape[0], _A), 1)
    row_empty = jnp.sum(xm, axis=1, keepdims=True) == 0
    xm0 = jnp.where((cols == 0) & row_empty, 1, xm)

    neg = jnp.float32(-jnp.inf)
    logits1 = jnp.where(xm0 != 0, out1, neg)
    m1 = jnp.max(logits1, axis=1, keepdims=True)
    idx1 = jnp.min(jnp.where(logits1 == m1, cols, _A), axis=1, keepdims=True)
    prob1 = (cols == idx1).astype(jnp.float32)

    xm1 = jnp.where(cols == idx1, 0, xm0)
    xm1 = jnp.where(idx1 == 0, 0, xm1)
    xm1 = jnp.where(cols == 0, 1, xm1)

    wsum = jnp.dot(prob1, emb_ref[...], preferred_element_type=jnp.float32)
    cur = rep + wsum
    gi2 = jnp.dot(cur, wih_ref[...], preferred_element_type=jnp.float32) + b_ih
    gh2 = jnp.dot(h1, whh_ref[...], preferred_element_type=jnp.float32) + b_hh
    r2 = jax.nn.sigmoid(gi2[:, :H] + gh2[:, :H])
    z2 = jax.nn.sigmoid(gi2[:, H:2 * H] + gh2[:, H:2 * H])
    n2 = jnp.tanh(gi2[:, 2 * H:] + r2 * gh2[:, 2 * H:])
    h2 = (1.0 - z2) * n2 + z2 * h1

    out2 = jnp.dot(h2, whead_ref[...], preferred_element_type=jnp.float32) + bhead_ref[...]
    logits2 = jnp.where(xm1 != 0, out2, neg)
    m2 = jnp.max(logits2, axis=1, keepdims=True)
    idx2 = jnp.min(jnp.where(logits2 == m2, cols, _A), axis=1, keepdims=True)
    prob2 = (cols == idx2).astype(jnp.float32)

    p1_ref[...] = prob1
    p2_ref[...] = prob2


def kernel(representation_emb, x_, W_ih, W_hh, b_ih, b_hh, W_head, b_head,
           atom_embedding):
    wihT = W_ih.T
    whhT = W_hh.T
    wheadT = W_head.T
    b_ih2 = b_ih.reshape(1, 3 * _H)
    b_hh2 = b_hh.reshape(1, 3 * _H)
    b_head2 = b_head.reshape(1, _A)

    grid = (_B // _R,)
    row_spec = lambda w: pl.BlockSpec((_R, w), lambda i: (i, 0))
    full = lambda a: pl.BlockSpec(a.shape, lambda i: (0,) * a.ndim)

    p1, p2 = pl.pallas_call(
        _body,
        grid=grid,
        in_specs=[
            row_spec(_H),
            row_spec(_A),
            full(wihT), full(whhT), full(b_ih2), full(b_hh2),
            full(wheadT), full(b_head2), full(atom_embedding),
        ],
        out_specs=[row_spec(_A), row_spec(_A)],
        out_shape=[
            jax.ShapeDtypeStruct((_B, _A), jnp.float32),
            jax.ShapeDtypeStruct((_B, _A), jnp.float32),
        ],
    )(representation_emb, x_, wihT, whhT, b_ih2, b_hh2, wheadT, b_head2,
      atom_embedding)

    return jnp.stack([p1, p2], axis=1)


# speedup vs baseline: 2.0413x; 1.1066x over previous
import jax
import jax.numpy as jnp
from jax.experimental import pallas as pl

_B = 8192
_A = 154
_H = 768
_R = 1024

_DNT = (((1,), (1,)), ((), ()))


def _body(rep_ref, x_ref, wih_ref, whh_ref, bih_ref, bhh_ref,
          whead_ref, bhead_ref, emb_ref, out_ref):
    rep = rep_ref[...]
    xm = x_ref[...]
    b_ih = bih_ref[...]
    b_hh = bhh_ref[...]
    H = _H

    gi = jax.lax.dot_general(rep, wih_ref[...], _DNT,
                             preferred_element_type=jnp.float32) + b_ih
    r = jax.nn.sigmoid(gi[:, :H] + b_hh[:, :H])
    z = jax.nn.sigmoid(gi[:, H:2 * H] + b_hh[:, H:2 * H])
    n = jnp.tanh(gi[:, 2 * H:] + r * b_hh[:, 2 * H:])
    h1 = (1.0 - z) * n

    out1 = jax.lax.dot_general(h1, whead_ref[...], _DNT,
                               preferred_element_type=jnp.float32) + bhead_ref[...]

    cols = jax.lax.broadcasted_iota(jnp.int32, (rep.shape[0], _A), 1)
    row_empty = jnp.sum(xm, axis=1, keepdims=True) == 0
    xm0 = jnp.where((cols == 0) & row_empty, 1, xm)

    neg = jnp.float32(-jnp.inf)
    logits1 = jnp.where(xm0 != 0, out1, neg)
    m1 = jnp.max(logits1, axis=1, keepdims=True)
    idx1 = jnp.min(jnp.where(logits1 == m1, cols, _A), axis=1, keepdims=True)
    prob1 = (cols == idx1).astype(jnp.float32)

    xm1 = jnp.where(cols == idx1, 0, xm0)
    xm1 = jnp.where(idx1 == 0, 0, xm1)
    xm1 = jnp.where(cols == 0, 1, xm1)

    wsum = jnp.dot(prob1, emb_ref[...], preferred_element_type=jnp.float32)
    cur = rep + wsum
    gi2 = jax.lax.dot_general(cur, wih_ref[...], _DNT,
                              preferred_element_type=jnp.float32) + b_ih
    gh2 = jax.lax.dot_general(h1, whh_ref[...], _DNT,
                              preferred_element_type=jnp.float32) + b_hh
    r2 = jax.nn.sigmoid(gi2[:, :H] + gh2[:, :H])
    z2 = jax.nn.sigmoid(gi2[:, H:2 * H] + gh2[:, H:2 * H])
    n2 = jnp.tanh(gi2[:, 2 * H:] + r2 * gh2[:, 2 * H:])
    h2 = (1.0 - z2) * n2 + z2 * h1

    out2 = jax.lax.dot_general(h2, whead_ref[...], _DNT,
                               preferred_element_type=jnp.float32) + bhead_ref[...]
    logits2 = jnp.where(xm1 != 0, out2, neg)
    m2 = jnp.max(logits2, axis=1, keepdims=True)
    idx2 = jnp.min(jnp.where(logits2 == m2, cols, _A), axis=1, keepdims=True)
    prob2 = (cols == idx2).astype(jnp.float32)

    out_ref[:, 0, :] = prob1
    out_ref[:, 1, :] = prob2


def kernel(representation_emb, x_, W_ih, W_hh, b_ih, b_hh, W_head, b_head,
           atom_embedding):
    b_ih2 = b_ih.reshape(1, 3 * _H)
    b_hh2 = b_hh.reshape(1, 3 * _H)
    b_head2 = b_head.reshape(1, _A)

    grid = (_B // _R,)
    row_spec = lambda w: pl.BlockSpec((_R, w), lambda i: (i, 0))
    full = lambda a: pl.BlockSpec(a.shape, lambda i: (0,) * a.ndim)

    out = pl.pallas_call(
        _body,
        grid=grid,
        in_specs=[
            row_spec(_H),
            row_spec(_A),
            full(W_ih), full(W_hh), full(b_ih2), full(b_hh2),
            full(W_head), full(b_head2), full(atom_embedding),
        ],
        out_specs=pl.BlockSpec((_R, 2, _A), lambda i: (i, 0, 0)),
        out_shape=jax.ShapeDtypeStruct((_B, 2, _A), jnp.float32),
    )(representation_emb, x_, W_ih, W_hh, b_ih2, b_hh2, W_head, b_head2,
      atom_embedding)

    return out


# speedup vs baseline: 2.3492x; 1.1508x over previous
import jax
import jax.numpy as jnp
from jax.experimental import pallas as pl

_B = 8192
_A = 154
_H = 768
_R = 1024

_DNT = (((1,), (1,)), ((), ()))


def _body(rep_ref, x_ref, wih_ref, whh_ref, bih_ref, bhh_ref,
          whead_ref, bhead_ref, emb_ref, idx_ref):
    rep = rep_ref[...]
    xm = x_ref[...]
    b_ih = bih_ref[...]
    b_hh = bhh_ref[...]
    H = _H

    gi = jax.lax.dot_general(rep, wih_ref[...], _DNT,
                             preferred_element_type=jnp.float32) + b_ih
    r = jax.nn.sigmoid(gi[:, :H] + b_hh[:, :H])
    z = jax.nn.sigmoid(gi[:, H:2 * H] + b_hh[:, H:2 * H])
    n = jnp.tanh(gi[:, 2 * H:] + r * b_hh[:, 2 * H:])
    h1 = (1.0 - z) * n

    out1 = jax.lax.dot_general(h1, whead_ref[...], _DNT,
                               preferred_element_type=jnp.float32) + bhead_ref[...]

    cols = jax.lax.broadcasted_iota(jnp.int32, (rep.shape[0], _A), 1)
    row_empty = jnp.max(xm.astype(jnp.int32), axis=1, keepdims=True) == 0
    xm0 = jnp.where((cols == 0) & row_empty, jnp.int8(1), xm)

    neg = jnp.float32(-jnp.inf)
    logits1 = jnp.where(xm0 != 0, out1, neg)
    m1 = jnp.max(logits1, axis=1, keepdims=True)
    idx1 = jnp.min(jnp.where(logits1 == m1, cols, _A), axis=1, keepdims=True)
    prob1 = (cols == idx1).astype(jnp.float32)

    xm1 = jnp.where(cols == idx1, jnp.int8(0), xm0)
    xm1 = jnp.where(idx1 == 0, jnp.int8(0), xm1)
    xm1 = jnp.where(cols == 0, jnp.int8(1), xm1)

    wsum = jnp.dot(prob1, emb_ref[...], preferred_element_type=jnp.float32)
    cur = rep + wsum
    gi2 = jax.lax.dot_general(cur, wih_ref[...], _DNT,
                              preferred_element_type=jnp.float32) + b_ih
    gh2 = jax.lax.dot_general(h1, whh_ref[...], _DNT,
                              preferred_element_type=jnp.float32) + b_hh
    r2 = jax.nn.sigmoid(gi2[:, :H] + gh2[:, :H])
    z2 = jax.nn.sigmoid(gi2[:, H:2 * H] + gh2[:, H:2 * H])
    n2 = jnp.tanh(gi2[:, 2 * H:] + r2 * gh2[:, 2 * H:])
    h2 = (1.0 - z2) * n2 + z2 * h1

    out2 = jax.lax.dot_general(h2, whead_ref[...], _DNT,
                               preferred_element_type=jnp.float32) + bhead_ref[...]
    logits2 = jnp.where(xm1 != 0, out2, neg)
    m2 = jnp.max(logits2, axis=1, keepdims=True)
    idx2 = jnp.min(jnp.where(logits2 == m2, cols, _A), axis=1, keepdims=True)

    idx_ref[:, 0:1] = idx1
    idx_ref[:, 1:2] = idx2


def kernel(representation_emb, x_, W_ih, W_hh, b_ih, b_hh, W_head, b_head,
           atom_embedding):
    b_ih2 = b_ih.reshape(1, 3 * _H)
    b_hh2 = b_hh.reshape(1, 3 * _H)
    b_head2 = b_head.reshape(1, _A)
    x8 = (x_ != 0).astype(jnp.int8)

    grid = (_B // _R,)
    row_spec = lambda w: pl.BlockSpec((_R, w), lambda i: (i, 0))
    full = lambda a: pl.BlockSpec(a.shape, lambda i: (0,) * a.ndim)

    idx = pl.pallas_call(
        _body,
        grid=grid,
        in_specs=[
            row_spec(_H),
            row_spec(_A),
            full(W_ih), full(W_hh), full(b_ih2), full(b_hh2),
            full(W_head), full(b_head2), full(atom_embedding),
        ],
        out_specs=pl.BlockSpec((_R, 2), lambda i: (i, 0)),
        out_shape=jax.ShapeDtypeStruct((_B, 2), jnp.int32),
    )(representation_emb, x8, W_ih, W_hh, b_ih2, b_hh2, W_head, b_head2,
      atom_embedding)

    cols = jax.lax.broadcasted_iota(jnp.int32, (1, 1, _A), 2)
    return (idx[:, :, None] == cols).astype(jnp.float32)
